# baseline (speedup 1.0000x reference)
import jax
import jax.numpy as jnp
from jax.experimental import pallas as pl
from jax.experimental.pallas import tpu as pltpu


def _copy_body(q_ref, o_ref):
    o_ref[...] = q_ref[...]


def kernel(c, q, emb_table):
    del c, emb_table
    rows, cols = q.shape
    qt = q.T
    grid = 16
    blk = qt.shape[1] // grid
    out_t = pl.pallas_call(
        _copy_body,
        grid=(grid,),
        in_specs=[pl.BlockSpec((cols, blk), lambda i: (0, i))],
        out_specs=pl.BlockSpec((cols, blk), lambda i: (0, i)),
        out_shape=jax.ShapeDtypeStruct((cols, rows), q.dtype),
        compiler_params=pltpu.CompilerParams(
            dimension_semantics=("parallel",)),
    )(qt)
    return out_t.T


# speedup vs baseline: 2.1658x; 2.1658x over previous
import jax
import jax.numpy as jnp
from jax.experimental import pallas as pl
from jax.experimental.pallas import tpu as pltpu


def _copy_body(q_ref, o_ref):
    o_ref[...] = q_ref[...]


def kernel(c, q, emb_table):
    del c, emb_table
    rows, cols = q.shape
    qt = q.T
    grid = 4
    blk = qt.shape[1] // grid
    out_t = pl.pallas_call(
        _copy_body,
        grid=(grid,),
        in_specs=[pl.BlockSpec((cols, blk), lambda i: (0, i))],
        out_specs=pl.BlockSpec((cols, blk), lambda i: (0, i)),
        out_shape=jax.ShapeDtypeStruct((cols, rows), q.dtype),
        compiler_params=pltpu.CompilerParams(
            dimension_semantics=("parallel",)),
    )(qt)
    return out_t.T
